# baseline (speedup 1.0000x reference)
import jax
import jax.numpy as jnp
from jax.experimental import pallas as pl
from jax.experimental.pallas import tpu as pltpu

N_EXPERTS = 16
TOP_K = 2
D_MODEL = 2048
N_TOKENS = 16384

BLK = 512


def _router_body(t_ref, x_ref, w_ref, rm_ref, idx_ref):
    inv_t = 1.0 / t_ref[0]
    lg = jax.lax.dot_general(
        w_ref[...], x_ref[...],
        dimension_numbers=(((1,), (1,)), ((), ())),
        preferred_element_type=jnp.float32,
    ) * inv_t
    e_iota = jax.lax.broadcasted_iota(jnp.int32, lg.shape, 0)
    big = jnp.int32(N_EXPERTS)
    m0 = jnp.max(lg, axis=0, keepdims=True)
    i0 = jnp.min(jnp.where(lg == m0, e_iota, big), axis=0, keepdims=True)
    masked = jnp.where(e_iota == i0, -jnp.inf, lg)
    m1 = jnp.max(masked, axis=0, keepdims=True)
    i1 = jnp.min(jnp.where(masked == m1, e_iota, big), axis=0, keepdims=True)
    e = jnp.exp(m1 - m0)
    w0 = 1.0 / (1.0 + e)
    w1 = e * w0
    rm_ref[...] = jnp.where(e_iota == i0, w0,
                            jnp.where(e_iota == i1, w1, jnp.float32(0.0)))
    idx_ref[...] = jnp.concatenate([i0, i1], axis=0)


def kernel(x, W, temperature):
    t = jnp.asarray(temperature, jnp.float32).reshape(1)
    rm_t, idx_t = pl.pallas_call(
        _router_body,
        grid=(N_TOKENS // BLK,),
        in_specs=[
            pl.BlockSpec(memory_space=pltpu.SMEM),
            pl.BlockSpec((BLK, D_MODEL), lambda i: (i, 0)),
            pl.BlockSpec((N_EXPERTS, D_MODEL), lambda i: (0, 0)),
        ],
        out_specs=[
            pl.BlockSpec((N_EXPERTS, BLK), lambda i: (0, i)),
            pl.BlockSpec((TOP_K, BLK), lambda i: (0, i)),
        ],
        out_shape=[
            jax.ShapeDtypeStruct((N_EXPERTS, N_TOKENS), jnp.float32),
            jax.ShapeDtypeStruct((TOP_K, N_TOKENS), jnp.int32),
        ],
        compiler_params=pltpu.CompilerParams(
            dimension_semantics=("arbitrary",),
        ),
    )(t, x, W)
    return (rm_t.T, idx_t.T)


# speedup vs baseline: 1.1862x; 1.1862x over previous
import jax
import jax.numpy as jnp
from jax.experimental import pallas as pl
from jax.experimental.pallas import tpu as pltpu

N_EXPERTS = 16
TOP_K = 2
D_MODEL = 2048
N_TOKENS = 16384

BLK = 1024


def _router_body(t_ref, x_ref, w_ref, rm_ref, idx_ref):
    inv_t = 1.0 / t_ref[0]
    lg = jax.lax.dot_general(
        w_ref[...], x_ref[...],
        dimension_numbers=(((1,), (1,)), ((), ())),
        preferred_element_type=jnp.float32,
    ) * inv_t
    e_iota = jax.lax.broadcasted_iota(jnp.int32, lg.shape, 0)
    big = jnp.int32(N_EXPERTS)
    m0 = jnp.max(lg, axis=0, keepdims=True)
    i0 = jnp.min(jnp.where(lg == m0, e_iota, big), axis=0, keepdims=True)
    masked = jnp.where(e_iota == i0, -jnp.inf, lg)
    m1 = jnp.max(masked, axis=0, keepdims=True)
    i1 = jnp.min(jnp.where(masked == m1, e_iota, big), axis=0, keepdims=True)
    e = jnp.exp(m1 - m0)
    w0 = 1.0 / (1.0 + e)
    w1 = e * w0
    rm_ref[...] = jnp.where(e_iota == i0, w0,
                            jnp.where(e_iota == i1, w1, jnp.float32(0.0)))
    idx_ref[...] = jnp.concatenate([i0, i1], axis=0)


def kernel(x, W, temperature):
    t = jnp.asarray(temperature, jnp.float32).reshape(1)
    rm_t, idx_t = pl.pallas_call(
        _router_body,
        grid=(N_TOKENS // BLK,),
        in_specs=[
            pl.BlockSpec(memory_space=pltpu.SMEM),
            pl.BlockSpec((BLK, D_MODEL), lambda i: (i, 0)),
            pl.BlockSpec((N_EXPERTS, D_MODEL), lambda i: (0, 0)),
        ],
        out_specs=[
            pl.BlockSpec((N_EXPERTS, BLK), lambda i: (0, i)),
            pl.BlockSpec((TOP_K, BLK), lambda i: (0, i)),
        ],
        out_shape=[
            jax.ShapeDtypeStruct((N_EXPERTS, N_TOKENS), jnp.float32),
            jax.ShapeDtypeStruct((TOP_K, N_TOKENS), jnp.int32),
        ],
        compiler_params=pltpu.CompilerParams(
            dimension_semantics=("arbitrary",),
        ),
    )(t, x, W)
    return (rm_t.T, idx_t.T)
